# Initial kernel scaffold; baseline (speedup 1.0000x reference)
import jax
import jax.numpy as jnp
from jax.experimental import pallas as pl


def kernel(x, w1, g1, b1, w2, g2, b2, wp):
    raise NotImplementedError("write your pallas kernel here")


# speedup vs baseline: 3.9848x; 3.9848x over previous
import functools

import jax
import jax.numpy as jnp
from jax.experimental import pallas as pl
from jax.experimental.pallas import tpu as pltpu


def _shift_stack(x):
    z = jnp.zeros((x.shape[0], 1), x.dtype)
    xl = jnp.concatenate([z, x[:, :-1]], axis=1)
    xr = jnp.concatenate([x[:, 1:], z], axis=1)
    return jnp.concatenate([xl, x, xr], axis=0)


def _pass1_kernel(x_ref, w1_ref, h1_ref, sum_ref, sq_ref, *, nb):
    w1c = w1_ref[...]
    for i in range(nb):
        xb = x_ref[i].astype(jnp.bfloat16)
        h1 = jnp.dot(w1c, _shift_stack(xb),
                     preferred_element_type=jnp.float32)
        h1_ref[i] = h1.astype(jnp.bfloat16)
        sum_ref[i] = jnp.sum(h1, axis=1, keepdims=True)
        sq_ref[i] = jnp.sum(h1 * h1, axis=1, keepdims=True)


def _pass2_kernel(h1_ref, w2_ref, s1_ref, t1_ref, h2_ref, sum_ref, sq_ref, *, nb):
    w2c = w2_ref[...]
    s1 = s1_ref[...]
    t1 = t1_ref[...]
    for i in range(nb):
        h1 = h1_ref[i].astype(jnp.float32)
        a1 = jnp.maximum(h1 * s1 + t1, 0.0).astype(jnp.bfloat16)
        h2 = jnp.dot(w2c, _shift_stack(a1),
                     preferred_element_type=jnp.float32)
        h2_ref[i] = h2.astype(jnp.bfloat16)
        sum_ref[i] = jnp.sum(h2, axis=1, keepdims=True)
        sq_ref[i] = jnp.sum(h2 * h2, axis=1, keepdims=True)


def _pass3_kernel(h2_ref, x_ref, wp_ref, s2_ref, t2_ref, o_ref, *, nb):
    wp = wp_ref[...]
    s2 = s2_ref[...]
    t2 = t2_ref[...]
    for i in range(nb):
        proj = jnp.dot(wp, x_ref[i].astype(jnp.bfloat16),
                       preferred_element_type=jnp.float32)
        z = h2_ref[i].astype(jnp.float32) * s2 + t2
        o_ref[i] = jnp.maximum(z + proj, 0.0)


def _finalize(sums, sqs, gamma, beta, count, eps):
    s = jnp.sum(sums[:, :, 0], axis=0)
    ss = jnp.sum(sqs[:, :, 0], axis=0)
    mean = s / count
    var = jnp.maximum(ss / count - mean * mean, 0.0)
    inv = jax.lax.rsqrt(var + eps)
    scale = gamma.astype(jnp.float32) * inv
    shift = beta.astype(jnp.float32) - mean * scale
    return scale[:, None], shift[:, None]


def kernel(x, w1, g1, b1, w2, g2, b2, wp, eps=1e-5):
    N, Cin, L = x.shape
    Cout = w1.shape[0]
    K = w1.shape[2]
    assert K == 3, "kernel specialized for k=3 'same' convolutions"

    w1c = jnp.transpose(w1, (0, 2, 1)).reshape(Cout, K * Cin).astype(jnp.bfloat16)
    w2c = jnp.transpose(w2, (0, 2, 1)).reshape(Cout, K * Cout).astype(jnp.bfloat16)
    wpc = wp[:, :, 0].astype(jnp.bfloat16)

    nb = next(n for n in (8, 4, 2, 1) if N % n == 0)
    grid = (N // nb,)
    cparams = pltpu.CompilerParams(
        dimension_semantics=("parallel",),
        vmem_limit_bytes=32 * 1024 * 1024,
    )

    def blk(c, l, dtype_shape=None):
        return pl.BlockSpec((nb, c, l), lambda i: (i, 0, 0))

    def rep(shape):
        return pl.BlockSpec(tuple(shape), lambda i: (0,) * len(shape))

    stat_shape = jax.ShapeDtypeStruct((N, Cout, 1), jnp.float32)
    stat_blk = pl.BlockSpec((nb, Cout, 1), lambda i: (i, 0, 0))

    h1, sum1, sq1 = pl.pallas_call(
        functools.partial(_pass1_kernel, nb=nb),
        grid=grid,
        in_specs=[blk(Cin, L), rep(w1c.shape)],
        out_specs=[blk(Cout, L), stat_blk, stat_blk],
        out_shape=[jax.ShapeDtypeStruct((N, Cout, L), jnp.bfloat16),
                   stat_shape, stat_shape],
        compiler_params=cparams,
    )(x, w1c)
    scale1, shift1 = _finalize(sum1, sq1, g1, b1, jnp.float32(N * L), eps)

    h2, sum2, sq2 = pl.pallas_call(
        functools.partial(_pass2_kernel, nb=nb),
        grid=grid,
        in_specs=[blk(Cout, L), rep(w2c.shape), rep((Cout, 1)), rep((Cout, 1))],
        out_specs=[blk(Cout, L), stat_blk, stat_blk],
        out_shape=[jax.ShapeDtypeStruct((N, Cout, L), jnp.bfloat16),
                   stat_shape, stat_shape],
        compiler_params=cparams,
    )(h1, w2c, scale1, shift1)
    scale2, shift2 = _finalize(sum2, sq2, g2, b2, jnp.float32(N * L), eps)

    out = pl.pallas_call(
        functools.partial(_pass3_kernel, nb=nb),
        grid=grid,
        in_specs=[blk(Cout, L), blk(Cin, L), rep(wpc.shape),
                  rep((Cout, 1)), rep((Cout, 1))],
        out_specs=blk(Cout, L),
        out_shape=jax.ShapeDtypeStruct((N, Cout, L), jnp.float32),
        compiler_params=cparams,
    )(h2, x, wpc, scale2, shift2)
    return out
